# Initial kernel scaffold; baseline (speedup 1.0000x reference)
import jax
import jax.numpy as jnp
from jax.experimental import pallas as pl


def kernel(prediction, bin_values, theta):
    raise NotImplementedError("write your pallas kernel here")


# speedup vs baseline: 2.3306x; 2.3306x over previous
import functools

import jax
import jax.numpy as jnp
from jax import lax
from jax.experimental import pallas as pl
from jax.experimental.pallas import tpu as pltpu
from jax.experimental.pallas import tpu_sc as plsc

_NB = 101
_L = 16
_NCHUNK = 7
_TAIL = _NB - (_NCHUNK - 1) * _L


def _count_less(bins_v, thresh_vec):
    lane = lax.iota(jnp.int32, _L)
    acc = jnp.zeros((_L,), jnp.int32)
    for k in range(_NCHUNK):
        chunk = bins_v[pl.ds(k * _L, _L)]
        lt = chunk < thresh_vec
        if k == _NCHUNK - 1:
            lt = jnp.logical_and(lt, lane < _TAIL)
        acc = acc + lt.astype(jnp.int32)
    return jnp.sum(acc)


def _gather_scalar(v_ref, idx_vec):
    lane = lax.iota(jnp.int32, _L)
    acc = jnp.zeros((_L,), jnp.float32)
    for k in range(_NCHUNK):
        chunk = v_ref[pl.ds(k * _L, _L)]
        acc = acc + jnp.where(lane + (k * _L) == idx_vec, chunk, 0.0)
    return jnp.sum(acc)


@functools.partial(
    pl.kernel,
    out_type=jax.ShapeDtypeStruct((1,), jnp.float32),
    mesh=plsc.VectorSubcoreMesh(core_axis_name="c", subcore_axis_name="s"),
    compiler_params=pltpu.CompilerParams(needs_layout_passes=False),
    scratch_types=[
        pltpu.VMEM((_L,), jnp.float32),
        pltpu.VMEM((_NCHUNK * _L,), jnp.float32),
        pltpu.VMEM((_NCHUNK * _L,), jnp.float32),
        pltpu.VMEM((_L,), jnp.float32),
    ],
)
def _sc_calibrate(pred_hbm, bins_hbm, theta_hbm, out_hbm,
                  pred_v, bins_v, theta_v, out_v):
    c = lax.axis_index("c")
    s = lax.axis_index("s")

    @pl.when(jnp.logical_and(c == 0, s == 0))
    def _():
        pltpu.sync_copy(pred_hbm, pred_v.at[pl.ds(0, 1)])
        pltpu.sync_copy(bins_hbm, bins_v.at[pl.ds(0, _NB)])
        pltpu.sync_copy(theta_hbm, theta_v.at[pl.ds(0, _NB)])

        lane = lax.iota(jnp.int32, _L)
        pred_s = jnp.sum(jnp.where(lane == 0, pred_v[...], 0.0))
        pred_vec = jnp.full((_L,), pred_s, jnp.float32)

        j = jnp.minimum(_count_less(bins_v, pred_vec), _NB - 1)
        j_vec = jnp.full((_L,), j, jnp.int32)
        adjusted = _gather_scalar(bins_v, j_vec) + _gather_scalar(theta_v, j_vec)
        adj_vec = jnp.full((_L,), adjusted, jnp.float32)

        i = jnp.minimum(_count_less(bins_v, adj_vec), _NB - 1)
        i_vec = jnp.full((_L,), i, jnp.int32)
        result = _gather_scalar(bins_v, i_vec)

        out_v[...] = jnp.full((_L,), result, jnp.float32)
        pltpu.sync_copy(out_v.at[pl.ds(0, 1)], out_hbm)


def kernel(prediction, bin_values, theta):
    pred1 = jnp.reshape(prediction, (1,))
    out = _sc_calibrate(pred1, bin_values, theta)
    return jnp.reshape(out, ())


# speedup vs baseline: 2.6356x; 1.1309x over previous
import functools

import jax
import jax.numpy as jnp
from jax import lax
from jax.experimental import pallas as pl
from jax.experimental.pallas import tpu as pltpu
from jax.experimental.pallas import tpu_sc as plsc

_NB = 101
_L = 16
_NCHUNK = 7
_TAIL = _NB - (_NCHUNK - 1) * _L


def _count_less(bins_v, thresh_vec):
    lane = lax.iota(jnp.int32, _L)
    acc = jnp.zeros((_L,), jnp.int32)
    for k in range(_NCHUNK):
        chunk = bins_v[pl.ds(k * _L, _L)]
        lt = chunk < thresh_vec
        if k == _NCHUNK - 1:
            lt = jnp.logical_and(lt, lane < _TAIL)
        acc = acc + lt.astype(jnp.int32)
    return jnp.sum(acc)


def _gather_scalar(v_ref, idx_vec):
    lane = lax.iota(jnp.int32, _L)
    acc = jnp.zeros((_L,), jnp.float32)
    for k in range(_NCHUNK):
        chunk = v_ref[pl.ds(k * _L, _L)]
        acc = acc + jnp.where(lane + (k * _L) == idx_vec, chunk, 0.0)
    return jnp.sum(acc)


@functools.partial(
    pl.kernel,
    out_type=jax.ShapeDtypeStruct((1,), jnp.float32),
    mesh=plsc.VectorSubcoreMesh(core_axis_name="c", subcore_axis_name="s",
                                num_cores=1, num_subcores=1),
    compiler_params=pltpu.CompilerParams(needs_layout_passes=False),
    scratch_types=[
        pltpu.VMEM((_L,), jnp.float32),
        pltpu.VMEM((_NCHUNK * _L,), jnp.float32),
        pltpu.VMEM((_NCHUNK * _L,), jnp.float32),
        pltpu.VMEM((_L,), jnp.float32),
        pltpu.SemaphoreType.DMA,
        pltpu.SemaphoreType.DMA,
        pltpu.SemaphoreType.DMA,
    ],
)
def _sc_calibrate(pred_hbm, bins_hbm, theta_hbm, out_hbm,
                  pred_v, bins_v, theta_v, out_v, sem_p, sem_b, sem_t):
    cp_p = pltpu.make_async_copy(pred_hbm, pred_v.at[pl.ds(0, 1)], sem_p)
    cp_b = pltpu.make_async_copy(bins_hbm, bins_v.at[pl.ds(0, _NB)], sem_b)
    cp_t = pltpu.make_async_copy(theta_hbm, theta_v.at[pl.ds(0, _NB)], sem_t)
    cp_p.start()
    cp_b.start()
    cp_t.start()
    cp_p.wait()
    cp_b.wait()

    lane = lax.iota(jnp.int32, _L)
    pred_s = jnp.sum(jnp.where(lane == 0, pred_v[...], 0.0))
    pred_vec = jnp.full((_L,), pred_s, jnp.float32)

    j = jnp.minimum(_count_less(bins_v, pred_vec), _NB - 1)
    j_vec = jnp.full((_L,), j, jnp.int32)
    cp_t.wait()
    adjusted = _gather_scalar(bins_v, j_vec) + _gather_scalar(theta_v, j_vec)
    adj_vec = jnp.full((_L,), adjusted, jnp.float32)

    i = jnp.minimum(_count_less(bins_v, adj_vec), _NB - 1)
    i_vec = jnp.full((_L,), i, jnp.int32)
    result = _gather_scalar(bins_v, i_vec)

    out_v[...] = jnp.full((_L,), result, jnp.float32)
    pltpu.sync_copy(out_v.at[pl.ds(0, 1)], out_hbm)


def kernel(prediction, bin_values, theta):
    pred1 = jnp.reshape(prediction, (1,))
    out = _sc_calibrate(pred1, bin_values, theta)
    return jnp.reshape(out, ())
